# baseline (speedup 1.0000x reference)
import functools

import jax
import jax.numpy as jnp
from jax import lax
from jax.experimental import pallas as pl
from jax.experimental.pallas import tpu as pltpu
from jax.experimental.pallas import tpu_sc as plsc

B = 4
T = 2048
D = 64
V = 1000000
NC = 2
NS = 16
NW = NC * NS
N = B * T
NG = N // 16
NBLK = (V + 127) // 128
BPW = (NBLK + NW - 1) // NW
LANES = 16
RING = 4
TRASH = N


def _emb_body(x_hbm, wt_hbm, wtp2_hbm, out_hbm,
              xall_v, hv_v, hp_v, flags_v, blist_v, colbuf_v,
              rowbuf_v, posbuf_v, pbuf_v, hc_s,
              sem0, sem1, sem2, sem3, scat_sem, pos_sem):
    fetch_sems = (sem0, sem1, sem2, sem3)
    cid = lax.axis_index("c")
    sid = lax.axis_index("s")
    wid = sid * NC + cid
    lo_b = wid * BPW
    lane_iota = lax.iota(jnp.int32, LANES)
    ones16 = jnp.ones((LANES,), jnp.int32)

    pltpu.sync_copy(x_hbm, xall_v)
    for g in range(BPW // LANES + 1):
        flags_v[pl.ds(g * LANES, LANES)] = jnp.zeros((LANES,), jnp.int32)
    hc_s[0] = 0
    pbuf_v[pl.ds(0, LANES)] = jnp.full((LANES,), TRASH, jnp.int32)
    pbuf_v[pl.ds(LANES, LANES)] = jnp.full((LANES,), TRASH, jnp.int32)

    def scan_g(g, nh):
        v16 = xall_v[pl.ds(g * LANES, LANES)]
        rel = (v16 >> 7) - lo_b
        m = (rel >= 0) & (rel < BPW)
        mi = m.astype(jnp.int32)
        slots = nh + jnp.cumsum(mi) - 1
        plsc.store_scatter(hv_v, [slots], v16, mask=m)
        plsc.store_scatter(hp_v, [slots], g * LANES + lane_iota, mask=m)
        plsc.store_scatter(flags_v, [rel], ones16, mask=m)
        return nh + jnp.sum(mi)

    nh = lax.fori_loop(0, NG, scan_g, 0)

    def bl_g(g, nb):
        f16 = flags_v[pl.ds(g * LANES, LANES)]
        slots = nb + jnp.cumsum(f16) - 1
        plsc.store_scatter(blist_v, [slots], g * LANES + lane_iota,
                           mask=f16 > 0)
        return nb + jnp.sum(f16)

    nb = lax.fori_loop(0, BPW // LANES + 1, bl_g, 0)
    nhg = (nh + LANES - 1) // LANES

    def vec_scalar(ref, j):
        grp = (j // LANES) * LANES
        v16 = ref[pl.ds(grp, LANES)]
        return lax.reduce_max(jnp.where(lane_iota == j - grp, v16, 0),
                              axes=(0,))

    def fire(j, r):
        bid = vec_scalar(blist_v, j)
        col = pl.multiple_of((lo_b + bid) * 128, 128)
        pltpu.async_copy(wt_hbm.at[:, pl.ds(col, 128)], colbuf_v.at[r],
                         fetch_sems[r])

    for r in range(RING):
        @pl.when(r < nb)
        def _(r=r):
            fire(r, r)

    def flush(parity):
        pbase = parity * LANES
        p16 = pbuf_v[pl.ds(pbase, LANES)]
        t16 = p16 & (T - 1)
        pltpu.async_copy(wtp2_hbm.at[t16 >> 1], posbuf_v, pos_sem).wait()
        for k in range(LANES):
            tk = lax.reduce_max(jnp.where(lane_iota == k, t16, 0), axes=(0,))
            poff = (tk & 1) * D
            for q in range(0, D, LANES):
                rowbuf_v[pbase + k, pl.ds(q, LANES)] = (
                    rowbuf_v[pbase + k, pl.ds(q, LANES)]
                    + posbuf_v[k, pl.ds(poff + q, LANES)])
        pltpu.async_copy(rowbuf_v.at[pl.ds(pbase, LANES)], out_hbm.at[p16],
                         scat_sem).wait()
        pbuf_v[pl.ds(pbase, LANES)] = jnp.full(
            (LANES,), TRASH, jnp.int32)

    def drain_hits(r, gb):
        def per_group(g, carry):
            v16 = hv_v[pl.ds(g * LANES, LANES)]
            p16 = hp_v[pl.ds(g * LANES, LANES)]
            live = (g * LANES + lane_iota) < nh
            m0 = ((v16 >> 7) == gb) & live

            def drain_cond(m):
                return jnp.any(m)

            def drain_one(m):
                l = lax.reduce_min(jnp.where(m, lane_iota, LANES), axes=(0,))
                sel = lane_iota == l
                v = lax.reduce_max(jnp.where(sel, v16, 0), axes=(0,))
                p = lax.reduce_max(jnp.where(sel, p16, 0), axes=(0,))
                hc = hc_s[0]
                slot = hc & (LANES - 1)
                parity = (hc >> 4) & 1
                pslot = parity * LANES + slot
                lane = jnp.full((LANES,), v & 127, jnp.int32)
                r16 = jnp.full((LANES,), r, jnp.int32)
                for q in range(0, D, LANES):
                    vals = plsc.load_gather(
                        colbuf_v, [r16, q + lane_iota, lane])
                    rowbuf_v[pslot, pl.ds(q, LANES)] = vals
                plsc.store_scatter(
                    pbuf_v, [jnp.full((LANES,), pslot, jnp.int32)],
                    jnp.full((LANES,), p, jnp.int32), mask=lane_iota < 1)
                hc_s[0] = hc + 1

                @pl.when(slot == LANES - 1)
                def _():
                    flush(parity)

                return m & jnp.logical_not(sel)

            lax.while_loop(drain_cond, drain_one, m0)
            return carry

        lax.fori_loop(0, nhg, per_group, 0)

    def do_block(blk, carry):
        for r in range(RING):
            j = blk * RING + r

            @pl.when(j < nb)
            def _(r=r, j=j):
                pltpu.make_async_copy(
                    wt_hbm.at[:, pl.ds(0, 128)], colbuf_v.at[r],
                    fetch_sems[r]).wait()
                gb = lo_b + vec_scalar(blist_v, j)
                drain_hits(r, gb)

                @pl.when(j + RING < nb)
                def _():
                    fire(j + RING, r)
        return carry

    lax.fori_loop(0, (nb + RING - 1) // RING, do_block, 0)

    @pl.when((hc_s[0] & (LANES - 1)) != 0)
    def _():
        flush((hc_s[0] >> 4) & 1)


@jax.jit
def _emb_lookup(x_flat, wt, wtp):
    mesh = plsc.VectorSubcoreMesh(core_axis_name="c", subcore_axis_name="s")
    return pl.kernel(
        _emb_body,
        out_type=jax.ShapeDtypeStruct((N + LANES, 2 * D), jnp.float32),
        mesh=mesh,
        scratch_types=[
            pltpu.VMEM((N,), jnp.int32),
            pltpu.VMEM((N,), jnp.int32),
            pltpu.VMEM((N,), jnp.int32),
            pltpu.VMEM((BPW + LANES,), jnp.int32),
            pltpu.VMEM((BPW + LANES,), jnp.int32),
            pltpu.VMEM((RING, D, 128), jnp.float32),
            pltpu.VMEM((2 * LANES, 2 * D), jnp.float32),
            pltpu.VMEM((LANES, 2 * D), jnp.float32),
            pltpu.VMEM((2 * LANES,), jnp.int32),
            pltpu.SMEM((1,), jnp.int32),
        ] + [pltpu.SemaphoreType.DMA] * (RING + 2),
        compiler_params=pltpu.CompilerParams(needs_layout_passes=False),
    )(x_flat, wt, wtp)


def kernel(x, wte, wtp):
    wide = _emb_lookup(x.reshape(-1), wte.T, wtp.reshape(T // 2, 2 * D))
    return wide[:N, :D].reshape(B, T, D)


# speedup vs baseline: 1.5505x; 1.5505x over previous
import functools

import jax
import jax.numpy as jnp
from jax import lax
from jax.experimental import pallas as pl
from jax.experimental.pallas import tpu as pltpu
from jax.experimental.pallas import tpu_sc as plsc

B = 4
T = 2048
D = 64
V = 1000000
NC = 2
NS = 16
NW = NC * NS
N = B * T
CHUNK = N // NW
LANES = 16
RING = 8


def _emb_body(x_hbm, wt_hbm, wtp_hbm, out_hbm,
              idx_v, colbuf_v, rows_v, pos_v, *sems):
    wid = lax.axis_index("s") * NC + lax.axis_index("c")
    base = wid * CHUNK
    b = base // T
    pos_off = lax.rem(base, T)

    pltpu.sync_copy(x_hbm.at[pl.ds(base, CHUNK)], idx_v)
    pltpu.sync_copy(wtp_hbm.at[:, pl.ds(pos_off, CHUNK)], pos_v)

    lane_iota = lax.iota(jnp.int32, LANES)

    def idx_scalar(i):
        v16 = idx_v[pl.ds((i // LANES) * LANES, LANES)]
        return lax.reduce_max(jnp.where(lane_iota == (i % LANES), v16, 0),
                              axes=(0,))

    def fire(i, r):
        v = idx_scalar(i)
        col = pl.multiple_of((v >> 7) * 128, 128)
        return pltpu.async_copy(
            wt_hbm.at[:, pl.ds(col, 128)], colbuf_v.at[r], sems[r])

    for r in range(RING):
        fire(r, r)

    def do_block(blk, carry):
        for r in range(RING):
            i = blk * RING + r
            pltpu.make_async_copy(
                wt_hbm.at[:, pl.ds(0, 128)], colbuf_v.at[r], sems[r]).wait()
            v = idx_scalar(i)
            i16 = jnp.full((LANES,), i, jnp.int32)
            lane = jnp.full((LANES,), v & 127, jnp.int32)
            for q in range(0, D, LANES):
                vals = plsc.load_gather(colbuf_v, [
                    jnp.full((LANES,), r, jnp.int32), q + lane_iota, lane])
                pvals = plsc.load_gather(pos_v, [q + lane_iota, i16])
                plsc.store_scatter(rows_v, [q + lane_iota, i16], vals + pvals)

            @pl.when(i + RING < CHUNK)
            def _():
                fire(i + RING, r)
        return carry

    lax.fori_loop(0, CHUNK // RING, do_block, 0)

    pltpu.sync_copy(rows_v, out_hbm.at[b, :, pl.ds(pos_off, CHUNK)])


@jax.jit
def _emb_lookup(x_flat, wt, wtp_t):
    mesh = plsc.VectorSubcoreMesh(core_axis_name="c", subcore_axis_name="s")
    return pl.kernel(
        _emb_body,
        out_type=jax.ShapeDtypeStruct((B, D, T), jnp.float32),
        mesh=mesh,
        scratch_types=[
            pltpu.VMEM((CHUNK,), jnp.int32),
            pltpu.VMEM((RING, D, 128), jnp.float32),
            pltpu.VMEM((D, CHUNK), jnp.float32),
            pltpu.VMEM((D, CHUNK), jnp.float32),
        ] + [pltpu.SemaphoreType.DMA] * RING,
        compiler_params=pltpu.CompilerParams(needs_layout_passes=False),
    )(x_flat, wt, wtp_t)


def kernel(x, wte, wtp):
    out_t = _emb_lookup(x.reshape(-1), wte.T, wtp.T)
    return out_t.transpose(0, 2, 1)
